# baseline (speedup 1.0000x reference)
import functools

import jax
import jax.numpy as jnp
from jax import lax
from jax.experimental import pallas as pl
from jax.experimental.pallas import tpu as pltpu
from jax.experimental.pallas import tpu_sc as plsc

NUM_TOKENS = 32768
NUM_EXPERTS = 64
TOPK = 8
LANES = 16

_NET8 = (
    (0, 1), (2, 3), (4, 5), (6, 7),
    (0, 2), (1, 3), (4, 6), (5, 7),
    (1, 2), (5, 6), (0, 4), (3, 7),
    (1, 5), (2, 6),
    (1, 4), (3, 6),
    (2, 4), (3, 5),
    (3, 4),
)
_BMERGE8 = (
    (0, 4), (1, 5), (2, 6), (3, 7),
    (0, 2), (1, 3), (4, 6), (5, 7),
    (0, 1), (2, 3), (4, 5), (6, 7),
)


def _ce(v, ix, i, j):
    c = v[i] < v[j]
    vi = jnp.where(c, v[j], v[i])
    vj = jnp.where(c, v[i], v[j])
    ii = jnp.where(c, ix[j], ix[i])
    ij = jnp.where(c, ix[i], ix[j])
    v[i], v[j], ix[i], ix[j] = vi, vj, ii, ij


def _sort8(v, ix):
    for (i, j) in _NET8:
        _ce(v, ix, i, j)


def _merge_top8(av, ai, bv, bi):
    lv, li = [None] * 8, [None] * 8
    for i in range(8):
        c = av[i] < bv[7 - i]
        lv[i] = jnp.where(c, bv[7 - i], av[i])
        li[i] = jnp.where(c, bi[7 - i], ai[i])
    for (i, j) in _BMERGE8:
        _ce(lv, li, i, j)
    return lv, li


def _top8_of_64(vals):
    acc_v = acc_i = None
    for g in range(8):
        gv = list(vals[8 * g:8 * g + 8])
        gi = [jnp.full((LANES,), 8 * g + j, jnp.int32) for j in range(8)]
        _sort8(gv, gi)
        if acc_v is None:
            acc_v, acc_i = gv, gi
        else:
            acc_v, acc_i = _merge_top8(acc_v, acc_i, gv, gi)
    return acc_v, acc_i


def _make_router():
    info = plsc.get_sparse_core_info()
    nc, ns = info.num_cores, info.num_subcores
    nw = nc * ns
    tpw = NUM_TOKENS // nw
    ngroups = tpw // LANES

    mesh = plsc.VectorSubcoreMesh(core_axis_name="c", subcore_axis_name="s")

    @functools.partial(
        pl.kernel,
        mesh=mesh,
        out_type=(
            jax.ShapeDtypeStruct((NUM_TOKENS * TOPK,), jnp.float32),
            jax.ShapeDtypeStruct((NUM_TOKENS * TOPK,), jnp.int32),
        ),
        scratch_types=[
            pltpu.VMEM((tpw * NUM_EXPERTS,), jnp.float32),
            pltpu.VMEM((tpw * TOPK,), jnp.float32),
            pltpu.VMEM((tpw * TOPK,), jnp.int32),
        ],
        compiler_params=pltpu.CompilerParams(needs_layout_passes=False),
    )
    def router(logits_hbm, w_hbm, i_hbm, logits_v, w_v, i_v):
        wid = lax.axis_index("s") * nc + lax.axis_index("c")
        base = wid * tpw
        pltpu.sync_copy(
            logits_hbm.at[pl.ds(base * NUM_EXPERTS, tpw * NUM_EXPERTS)],
            logits_v)

        lane = lax.iota(jnp.int32, LANES)

        def body(g, carry):
            row = g * LANES + lane
            rowl = row * NUM_EXPERTS
            vals = [
                plsc.load_gather(logits_v, [rowl + e])
                for e in range(NUM_EXPERTS)
            ]
            top_v, top_i = _top8_of_64(vals)
            es = [jnp.exp(v - top_v[0]) for v in top_v]
            total = es[0]
            for j in range(1, 8):
                total = total + es[j]
            inv = 1.0 / total
            rowk = row * TOPK
            for j in range(8):
                plsc.store_scatter(w_v, [rowk + j], es[j] * inv)
                plsc.store_scatter(i_v, [rowk + j], top_i[j])
            return carry

        lax.fori_loop(0, ngroups, body, 0)

        pltpu.sync_copy(w_v, w_hbm.at[pl.ds(base * TOPK, tpw * TOPK)])
        pltpu.sync_copy(i_v, i_hbm.at[pl.ds(base * TOPK, tpw * TOPK)])

    return router


@functools.cache
def _get_router():
    return _make_router()


def kernel(gate_logits):
    weights, indices = _get_router()(gate_logits.reshape(-1))
    return (weights.reshape(NUM_TOKENS, TOPK),
            indices.reshape(NUM_TOKENS, TOPK))


# speedup vs baseline: 1.0027x; 1.0027x over previous
import functools

import jax
import jax.numpy as jnp
from jax import lax
from jax.experimental import pallas as pl
from jax.experimental.pallas import tpu as pltpu
from jax.experimental.pallas import tpu_sc as plsc

NUM_TOKENS = 32768
NUM_EXPERTS = 64
TOPK = 8
LANES = 16

_NET8 = (
    (0, 1), (2, 3), (4, 5), (6, 7),
    (0, 2), (1, 3), (4, 6), (5, 7),
    (1, 2), (5, 6), (0, 4), (3, 7),
    (1, 5), (2, 6),
    (1, 4), (3, 6),
    (2, 4), (3, 5),
    (3, 4),
)
_BMERGE8 = (
    (0, 4), (1, 5), (2, 6), (3, 7),
    (0, 2), (1, 3), (4, 6), (5, 7),
    (0, 1), (2, 3), (4, 5), (6, 7),
)


def _ce(v, ix, i, j):
    c = v[i] < v[j]
    vi = jnp.where(c, v[j], v[i])
    vj = jnp.where(c, v[i], v[j])
    ii = jnp.where(c, ix[j], ix[i])
    ij = jnp.where(c, ix[i], ix[j])
    v[i], v[j], ix[i], ix[j] = vi, vj, ii, ij


def _sort8(v, ix):
    for (i, j) in _NET8:
        _ce(v, ix, i, j)


def _merge_top8(av, ai, bv, bi):
    lv, li = [None] * 8, [None] * 8
    for i in range(8):
        c = av[i] < bv[7 - i]
        lv[i] = jnp.where(c, bv[7 - i], av[i])
        li[i] = jnp.where(c, bi[7 - i], ai[i])
    for (i, j) in _BMERGE8:
        _ce(lv, li, i, j)
    return lv, li


def _top8_of_64(vals):
    acc_v = acc_i = None
    for g in range(8):
        gv = list(vals[8 * g:8 * g + 8])
        gi = [jnp.full((LANES,), 8 * g + j, jnp.int32) for j in range(8)]
        _sort8(gv, gi)
        if acc_v is None:
            acc_v, acc_i = gv, gi
        else:
            acc_v, acc_i = _merge_top8(acc_v, acc_i, gv, gi)
    return acc_v, acc_i


def _make_router():
    info = plsc.get_sparse_core_info()
    nc, ns = info.num_cores, info.num_subcores
    nw = nc * ns
    tpw = NUM_TOKENS // nw
    ngroups = tpw // LANES

    mesh = plsc.VectorSubcoreMesh(core_axis_name="c", subcore_axis_name="s")

    @functools.partial(
        pl.kernel,
        mesh=mesh,
        out_type=(
            jax.ShapeDtypeStruct((NUM_TOKENS, TOPK), jnp.float32),
            jax.ShapeDtypeStruct((NUM_TOKENS, TOPK), jnp.int32),
        ),
        scratch_types=[
            pltpu.VMEM((tpw, NUM_EXPERTS), jnp.float32),
            pltpu.VMEM((tpw, TOPK), jnp.float32),
            pltpu.VMEM((tpw, TOPK), jnp.int32),
        ],
        compiler_params=pltpu.CompilerParams(
            needs_layout_passes=False, use_tc_tiling_on_sc=False),
    )
    def router(logits_hbm, w_hbm, i_hbm, logits_v, w_v, i_v):
        wid = lax.axis_index("s") * nc + lax.axis_index("c")
        base = wid * tpw
        pltpu.sync_copy(logits_hbm.at[pl.ds(base, tpw)], logits_v)

        lane = lax.iota(jnp.int32, LANES)

        def body(g, carry):
            row = g * LANES + lane
            vals = [
                plsc.load_gather(
                    logits_v, [row, jnp.full((LANES,), e, jnp.int32)])
                for e in range(NUM_EXPERTS)
            ]
            top_v, top_i = _top8_of_64(vals)
            es = [jnp.exp(v - top_v[0]) for v in top_v]
            total = es[0]
            for j in range(1, 8):
                total = total + es[j]
            inv = 1.0 / total
            for j in range(8):
                col = jnp.full((LANES,), j, jnp.int32)
                plsc.store_scatter(w_v, [row, col], es[j] * inv)
                plsc.store_scatter(i_v, [row, col], top_i[j])
            return carry

        lax.fori_loop(0, ngroups, body, 0)

        pltpu.sync_copy(w_v, w_hbm.at[pl.ds(base, tpw)])
        pltpu.sync_copy(i_v, i_hbm.at[pl.ds(base, tpw)])

    return router


@functools.cache
def _get_router():
    return _make_router()


def kernel(gate_logits):
    weights, indices = _get_router()(gate_logits)
    return (weights, indices)


# speedup vs baseline: 2.6171x; 2.6101x over previous
import functools

import jax
import jax.numpy as jnp
from jax import lax
from jax.experimental import pallas as pl
from jax.experimental.pallas import tpu as pltpu
from jax.experimental.pallas import tpu_sc as plsc

NUM_TOKENS = 32768
NUM_EXPERTS = 64
TOPK = 8
LANES = 16
TBLK = 128
EBLK = 8
NEB = NUM_EXPERTS // EBLK
NTB = NUM_TOKENS // TBLK

_NET8 = (
    (0, 1), (2, 3), (4, 5), (6, 7),
    (0, 2), (1, 3), (4, 6), (5, 7),
    (1, 2), (5, 6), (0, 4), (3, 7),
    (1, 5), (2, 6),
    (1, 4), (3, 6),
    (2, 4), (3, 5),
    (3, 4),
)
_BMERGE8 = (
    (0, 4), (1, 5), (2, 6), (3, 7),
    (0, 2), (1, 3), (4, 6), (5, 7),
    (0, 1), (2, 3), (4, 5), (6, 7),
)


def _ce(v, ix, i, j):
    c = v[i] < v[j]
    vi = jnp.where(c, v[j], v[i])
    vj = jnp.where(c, v[i], v[j])
    ii = jnp.where(c, ix[j], ix[i])
    ij = jnp.where(c, ix[i], ix[j])
    v[i], v[j], ix[i], ix[j] = vi, vj, ii, ij


def _sort8(v, ix):
    for (i, j) in _NET8:
        _ce(v, ix, i, j)


def _merge_top8(av, ai, bv, bi):
    lv, li = [None] * 8, [None] * 8
    for i in range(8):
        c = av[i] < bv[7 - i]
        lv[i] = jnp.where(c, bv[7 - i], av[i])
        li[i] = jnp.where(c, bi[7 - i], ai[i])
    for (i, j) in _BMERGE8:
        _ce(lv, li, i, j)
    return lv, li


def _top8_of_64(vals):
    acc_v = acc_i = None
    for g in range(8):
        gv = list(vals[8 * g:8 * g + 8])
        gi = [jnp.full((LANES,), 8 * g + j, jnp.int32) for j in range(8)]
        _sort8(gv, gi)
        if acc_v is None:
            acc_v, acc_i = gv, gi
        else:
            acc_v, acc_i = _merge_top8(acc_v, acc_i, gv, gi)
    return acc_v, acc_i


def _make_router():
    info = plsc.get_sparse_core_info()
    nc, ns = info.num_cores, info.num_subcores
    nw = nc * ns
    tpw = NUM_TOKENS // nw
    tbpw = tpw // TBLK
    ngroups = tpw // LANES

    mesh = plsc.VectorSubcoreMesh(core_axis_name="c", subcore_axis_name="s")

    @functools.partial(
        pl.kernel,
        mesh=mesh,
        out_type=(
            jax.ShapeDtypeStruct((NTB, TOPK, TBLK), jnp.float32),
            jax.ShapeDtypeStruct((NTB, TOPK, TBLK), jnp.int32),
        ),
        scratch_types=[
            pltpu.VMEM((NEB, tbpw, EBLK, TBLK), jnp.float32),
            pltpu.VMEM((tbpw, TOPK, TBLK), jnp.float32),
            pltpu.VMEM((tbpw, TOPK, TBLK), jnp.int32),
        ],
        compiler_params=pltpu.CompilerParams(
            needs_layout_passes=False, use_tc_tiling_on_sc=False),
    )
    def router(logits_hbm, w_hbm, i_hbm, logits_v, w_v, i_v):
        wid = lax.axis_index("s") * nc + lax.axis_index("c")
        tb0 = wid * tbpw
        for eb in range(NEB):
            pltpu.sync_copy(logits_hbm.at[eb, pl.ds(tb0, tbpw)],
                            logits_v.at[eb])

        def body(g, carry):
            tb = g // (TBLK // LANES)
            tr0 = (g % (TBLK // LANES)) * LANES
            vals = [
                logits_v[e // EBLK, tb, e % EBLK, pl.ds(tr0, LANES)]
                for e in range(NUM_EXPERTS)
            ]
            top_v, top_i = _top8_of_64(vals)
            es = [jnp.exp(v - top_v[0]) for v in top_v]
            total = es[0]
            for j in range(1, 8):
                total = total + es[j]
            inv = 1.0 / total
            for j in range(8):
                w_v[tb, j, pl.ds(tr0, LANES)] = es[j] * inv
                i_v[tb, j, pl.ds(tr0, LANES)] = top_i[j]
            return carry

        lax.fori_loop(0, ngroups, body, 0)

        pltpu.sync_copy(w_v, w_hbm.at[pl.ds(tb0, tbpw)])
        pltpu.sync_copy(i_v, i_hbm.at[pl.ds(tb0, tbpw)])

    return router


@functools.cache
def _get_router():
    return _make_router()


def kernel(gate_logits):
    x4 = (gate_logits.T
          .reshape(NEB, EBLK, NTB, TBLK)
          .transpose(0, 2, 1, 3))
    w3, i3 = _get_router()(x4)
    weights = w3.transpose(0, 2, 1).reshape(NUM_TOKENS, TOPK)
    indices = i3.transpose(0, 2, 1).reshape(NUM_TOKENS, TOPK)
    return (weights, indices)


# speedup vs baseline: 3.2169x; 1.2292x over previous
import functools

import jax
import jax.numpy as jnp
from jax import lax
from jax.experimental import pallas as pl
from jax.experimental.pallas import tpu as pltpu
from jax.experimental.pallas import tpu_sc as plsc

NUM_TOKENS = 32768
NUM_EXPERTS = 64
TOPK = 8
LANES = 16
TBLK = 128
EBLK = 8
NEB = NUM_EXPERTS // EBLK
NTB = NUM_TOKENS // TBLK

_NET8 = (
    (0, 1), (2, 3), (4, 5), (6, 7),
    (0, 2), (1, 3), (4, 6), (5, 7),
    (1, 2), (5, 6), (0, 4), (3, 7),
    (1, 5), (2, 6),
    (1, 4), (3, 6),
    (2, 4), (3, 5),
    (3, 4),
)
_BMERGE8 = (
    (0, 4), (1, 5), (2, 6), (3, 7),
    (0, 2), (1, 3), (4, 6), (5, 7),
    (0, 1), (2, 3), (4, 5), (6, 7),
)


def _top8_keys(keys):
    acc = None
    for g in range(8):
        gv = list(keys[8 * g:8 * g + 8])
        for (i, j) in _NET8:
            hi = jnp.maximum(gv[i], gv[j])
            lo = jnp.minimum(gv[i], gv[j])
            gv[i], gv[j] = hi, lo
        if acc is None:
            acc = gv
        else:
            lv = [jnp.maximum(acc[i], gv[7 - i]) for i in range(8)]
            for (i, j) in _BMERGE8:
                hi = jnp.maximum(lv[i], lv[j])
                lo = jnp.minimum(lv[i], lv[j])
                lv[i], lv[j] = hi, lo
            acc = lv
    return acc


def _lex_ce(v, ix, i, j):
    c = (v[i] < v[j]) | ((v[i] == v[j]) & (ix[i] > ix[j]))
    vi = jnp.where(c, v[j], v[i])
    vj = jnp.where(c, v[i], v[j])
    ii = jnp.where(c, ix[j], ix[i])
    ij = jnp.where(c, ix[i], ix[j])
    v[i], v[j], ix[i], ix[j] = vi, vj, ii, ij


def _make_router():
    info = plsc.get_sparse_core_info()
    nc, ns = info.num_cores, info.num_subcores
    nw = nc * ns
    tpw = NUM_TOKENS // nw
    tbpw = tpw // TBLK
    ngroups = tpw // LANES

    mesh = plsc.VectorSubcoreMesh(core_axis_name="c", subcore_axis_name="s")

    @functools.partial(
        pl.kernel,
        mesh=mesh,
        out_type=(
            jax.ShapeDtypeStruct((NTB, TOPK, TBLK), jnp.float32),
            jax.ShapeDtypeStruct((NTB, TOPK, TBLK), jnp.int32),
        ),
        scratch_types=[
            pltpu.VMEM((NEB, tbpw, EBLK, TBLK), jnp.float32),
            pltpu.VMEM((tbpw, TOPK, TBLK), jnp.float32),
            pltpu.VMEM((tbpw, TOPK, TBLK), jnp.int32),
        ],
        compiler_params=pltpu.CompilerParams(
            needs_layout_passes=False, use_tc_tiling_on_sc=False),
    )
    def router(logits_hbm, w_hbm, i_hbm, logits_v, w_v, i_v):
        wid = lax.axis_index("s") * nc + lax.axis_index("c")
        tb0 = wid * tbpw
        for eb in range(NEB):
            pltpu.sync_copy(logits_hbm.at[eb, pl.ds(tb0, tbpw)],
                            logits_v.at[eb])

        lane = lax.iota(jnp.int32, LANES)

        def body(g, carry):
            tb = g // (TBLK // LANES)
            tr0 = (g % (TBLK // LANES)) * LANES
            keys = []
            for e in range(NUM_EXPERTS):
                v = logits_v[e // EBLK, tb, e % EBLK, pl.ds(tr0, LANES)]
                kb = (lax.bitcast_convert_type(v, jnp.int32)
                      & jnp.int32(-64)) | jnp.int32(e)
                keys.append(lax.bitcast_convert_type(kb, jnp.float32))
            top = _top8_keys(keys)
            ids = [lax.bitcast_convert_type(k, jnp.int32) & jnp.int32(63)
                   for k in top]
            tbv = jnp.full((LANES,), tb, jnp.int32)
            trv = tr0 + lane
            vals = [
                plsc.load_gather(
                    logits_v,
                    [ids[j] >> 3, tbv, ids[j] & jnp.int32(7), trv])
                for j in range(TOPK)
            ]
            for (i, j) in _NET8:
                _lex_ce(vals, ids, i, j)
            es = [jnp.exp(v - vals[0]) for v in vals]
            total = es[0]
            for j in range(1, TOPK):
                total = total + es[j]
            inv = 1.0 / total
            for j in range(TOPK):
                w_v[tb, j, pl.ds(tr0, LANES)] = es[j] * inv
                i_v[tb, j, pl.ds(tr0, LANES)] = ids[j]
            return carry

        lax.fori_loop(0, ngroups, body, 0)

        pltpu.sync_copy(w_v, w_hbm.at[pl.ds(tb0, tbpw)])
        pltpu.sync_copy(i_v, i_hbm.at[pl.ds(tb0, tbpw)])

    return router


@functools.cache
def _get_router():
    return _make_router()


def kernel(gate_logits):
    x4 = (gate_logits.T
          .reshape(NEB, EBLK, NTB, TBLK)
          .transpose(0, 2, 1, 3))
    w3, i3 = _get_router()(x4)
    weights = w3.transpose(0, 2, 1).reshape(NUM_TOKENS, TOPK)
    indices = i3.transpose(0, 2, 1).reshape(NUM_TOKENS, TOPK)
    return (weights, indices)


# speedup vs baseline: 3.5102x; 1.0912x over previous
import functools

import jax
import jax.numpy as jnp
from jax import lax
from jax.experimental import pallas as pl
from jax.experimental.pallas import tpu as pltpu
from jax.experimental.pallas import tpu_sc as plsc

NUM_TOKENS = 32768
NUM_EXPERTS = 64
TOPK = 8
LANES = 16
TBLK = 128
EBLK = 8
NEB = NUM_EXPERTS // EBLK
NTB = NUM_TOKENS // TBLK

_NET8 = (
    (0, 1), (2, 3), (4, 5), (6, 7),
    (0, 2), (1, 3), (4, 6), (5, 7),
    (1, 2), (5, 6), (0, 4), (3, 7),
    (1, 5), (2, 6),
    (1, 4), (3, 6),
    (2, 4), (3, 5),
    (3, 4),
)
_BMERGE8 = (
    (0, 4), (1, 5), (2, 6), (3, 7),
    (0, 2), (1, 3), (4, 6), (5, 7),
    (0, 1), (2, 3), (4, 5), (6, 7),
)


def _top8_keys(keys):
    acc = None
    for g in range(8):
        gv = list(keys[8 * g:8 * g + 8])
        for (i, j) in _NET8:
            hi = jnp.maximum(gv[i], gv[j])
            lo = jnp.minimum(gv[i], gv[j])
            gv[i], gv[j] = hi, lo
        if acc is None:
            acc = gv
        else:
            lv = [jnp.maximum(acc[i], gv[7 - i]) for i in range(8)]
            for (i, j) in _BMERGE8:
                hi = jnp.maximum(lv[i], lv[j])
                lo = jnp.minimum(lv[i], lv[j])
                lv[i], lv[j] = hi, lo
            acc = lv
    return acc


def _lex_ce(v, ix, i, j):
    c = (v[i] < v[j]) | ((v[i] == v[j]) & (ix[i] > ix[j]))
    vi = jnp.where(c, v[j], v[i])
    vj = jnp.where(c, v[i], v[j])
    ii = jnp.where(c, ix[j], ix[i])
    ij = jnp.where(c, ix[i], ix[j])
    v[i], v[j], ix[i], ix[j] = vi, vj, ii, ij


def _make_router():
    info = plsc.get_sparse_core_info()
    nc, ns = info.num_cores, info.num_subcores
    nw = nc * ns
    tpw = NUM_TOKENS // nw
    tbpw = tpw // TBLK
    ngroups = tpw // LANES

    mesh = plsc.VectorSubcoreMesh(core_axis_name="c", subcore_axis_name="s")

    @functools.partial(
        pl.kernel,
        mesh=mesh,
        out_type=(
            jax.ShapeDtypeStruct((NTB, TOPK, TBLK), jnp.float32),
            jax.ShapeDtypeStruct((NTB, TOPK, TBLK), jnp.int32),
        ),
        scratch_types=[
            pltpu.VMEM((NEB, tbpw, EBLK, TBLK), jnp.float32),
            pltpu.VMEM((tbpw, TOPK, TBLK), jnp.float32),
            pltpu.VMEM((tbpw, TOPK, TBLK), jnp.int32),
            pltpu.SemaphoreType.DMA,
            pltpu.SemaphoreType.DMA,
        ],
        compiler_params=pltpu.CompilerParams(
            needs_layout_passes=False, use_tc_tiling_on_sc=False),
    )
    def router(logits_hbm, w_hbm, i_hbm, logits_v, w_v, i_v, sem_a, sem_b):
        wid = lax.axis_index("s") * nc + lax.axis_index("c")
        tb0 = wid * tbpw
        half = tbpw // 2
        copy_a = pltpu.async_copy(
            logits_hbm.at[:, pl.ds(tb0, half)],
            logits_v.at[:, pl.ds(0, half)], sem_a)
        copy_b = pltpu.async_copy(
            logits_hbm.at[:, pl.ds(tb0 + half, half)],
            logits_v.at[:, pl.ds(half, half)], sem_b)

        lane = lax.iota(jnp.int32, LANES)

        def body(g, carry):
            tb = g // (TBLK // LANES)
            tr0 = (g % (TBLK // LANES)) * LANES
            keys = []
            for e in range(NUM_EXPERTS):
                v = logits_v[e // EBLK, tb, e % EBLK, pl.ds(tr0, LANES)]
                kb = (lax.bitcast_convert_type(v, jnp.int32)
                      & jnp.int32(-64)) | jnp.int32(e)
                keys.append(lax.bitcast_convert_type(kb, jnp.float32))
            top = _top8_keys(keys)
            ids = [lax.bitcast_convert_type(k, jnp.int32) & jnp.int32(63)
                   for k in top]
            tbv = jnp.full((LANES,), tb, jnp.int32)
            trv = tr0 + lane
            vals = [
                plsc.load_gather(
                    logits_v,
                    [ids[j] >> 3, tbv, ids[j] & jnp.int32(7), trv])
                for j in range(TOPK)
            ]
            for (i, j) in _NET8:
                _lex_ce(vals, ids, i, j)
            es = ([jnp.full((LANES,), 1.0, jnp.float32)]
                  + [jnp.exp(v - vals[0]) for v in vals[1:]])
            total = es[0]
            for j in range(1, TOPK):
                total = total + es[j]
            inv = 1.0 / total
            for j in range(TOPK):
                w_v[tb, j, pl.ds(tr0, LANES)] = es[j] * inv
                i_v[tb, j, pl.ds(tr0, LANES)] = ids[j]
            return carry

        copy_a.wait()
        lax.fori_loop(0, ngroups // 2, body, 0)
        copy_b.wait()
        lax.fori_loop(ngroups // 2, ngroups, body, 0)

        pltpu.sync_copy(w_v, w_hbm.at[pl.ds(tb0, tbpw)])
        pltpu.sync_copy(i_v, i_hbm.at[pl.ds(tb0, tbpw)])

    return router


@functools.cache
def _get_router():
    return _make_router()


def kernel(gate_logits):
    x4 = (gate_logits.T
          .reshape(NEB, EBLK, NTB, TBLK)
          .transpose(0, 2, 1, 3))
    w3, i3 = _get_router()(x4)
    weights = w3.transpose(0, 2, 1).reshape(NUM_TOKENS, TOPK)
    indices = i3.transpose(0, 2, 1).reshape(NUM_TOKENS, TOPK)
    return (weights, indices)


# speedup vs baseline: 3.8030x; 1.0834x over previous
import functools

import jax
import jax.numpy as jnp
from jax import lax
from jax.experimental import pallas as pl
from jax.experimental.pallas import tpu as pltpu
from jax.experimental.pallas import tpu_sc as plsc

NUM_TOKENS = 32768
NUM_EXPERTS = 64
TOPK = 8
LANES = 16
TBLK = 128
EBLK = 8
NEB = NUM_EXPERTS // EBLK
NTB = NUM_TOKENS // TBLK

_NET8 = (
    (0, 1), (2, 3), (4, 5), (6, 7),
    (0, 2), (1, 3), (4, 6), (5, 7),
    (1, 2), (5, 6), (0, 4), (3, 7),
    (1, 5), (2, 6),
    (1, 4), (3, 6),
    (2, 4), (3, 5),
    (3, 4),
)
_BMERGE8 = (
    (0, 4), (1, 5), (2, 6), (3, 7),
    (0, 2), (1, 3), (4, 6), (5, 7),
    (0, 1), (2, 3), (4, 5), (6, 7),
)


def _top8_keys(keys):
    acc = None
    for g in range(8):
        gv = list(keys[8 * g:8 * g + 8])
        for (i, j) in _NET8:
            hi = jnp.maximum(gv[i], gv[j])
            lo = jnp.minimum(gv[i], gv[j])
            gv[i], gv[j] = hi, lo
        if acc is None:
            acc = gv
        else:
            lv = [jnp.maximum(acc[i], gv[7 - i]) for i in range(8)]
            for (i, j) in _BMERGE8:
                hi = jnp.maximum(lv[i], lv[j])
                lo = jnp.minimum(lv[i], lv[j])
                lv[i], lv[j] = hi, lo
            acc = lv
    return acc


def _make_router():
    info = plsc.get_sparse_core_info()
    nc, ns = info.num_cores, info.num_subcores
    nw = nc * ns
    tpw = NUM_TOKENS // nw
    tbpw = tpw // TBLK
    ngroups = tpw // LANES

    mesh = plsc.VectorSubcoreMesh(core_axis_name="c", subcore_axis_name="s")

    @functools.partial(
        pl.kernel,
        mesh=mesh,
        out_type=(
            jax.ShapeDtypeStruct((NTB, TOPK, TBLK), jnp.float32),
            jax.ShapeDtypeStruct((NTB, TOPK, TBLK), jnp.int32),
        ),
        scratch_types=[
            pltpu.VMEM((NEB, tbpw, EBLK, TBLK), jnp.float32),
            pltpu.VMEM((tbpw, TOPK, TBLK), jnp.float32),
            pltpu.VMEM((tbpw, TOPK, TBLK), jnp.int32),
            pltpu.SemaphoreType.DMA,
            pltpu.SemaphoreType.DMA,
        ],
        compiler_params=pltpu.CompilerParams(
            needs_layout_passes=False, use_tc_tiling_on_sc=False),
    )
    def router(logits_hbm, w_hbm, i_hbm, logits_v, w_v, i_v, sem_a, sem_b):
        wid = lax.axis_index("s") * nc + lax.axis_index("c")
        tb0 = wid * tbpw
        half = tbpw // 2
        copy_a = pltpu.async_copy(
            logits_hbm.at[:, pl.ds(tb0, half)],
            logits_v.at[:, pl.ds(0, half)], sem_a)
        copy_b = pltpu.async_copy(
            logits_hbm.at[:, pl.ds(tb0 + half, half)],
            logits_v.at[:, pl.ds(half, half)], sem_b)

        def body(g, carry):
            tb = g // (TBLK // LANES)
            tr0 = (g % (TBLK // LANES)) * LANES
            keys = []
            for e in range(NUM_EXPERTS):
                v = logits_v[e // EBLK, tb, e % EBLK, pl.ds(tr0, LANES)]
                kb = (lax.bitcast_convert_type(v, jnp.int32)
                      & jnp.int32(-64)) | jnp.int32(63 - e)
                keys.append(lax.bitcast_convert_type(kb, jnp.float32))
            top = _top8_keys(keys)
            ids = [63 - (lax.bitcast_convert_type(k, jnp.int32)
                         & jnp.int32(63))
                   for k in top]
            es = ([jnp.full((LANES,), 1.0, jnp.float32)]
                  + [jnp.exp(v - top[0]) for v in top[1:]])
            total = es[0]
            for j in range(1, TOPK):
                total = total + es[j]
            inv = 1.0 / total
            for j in range(TOPK):
                w_v[tb, j, pl.ds(tr0, LANES)] = es[j] * inv
                i_v[tb, j, pl.ds(tr0, LANES)] = ids[j]
            return carry

        copy_a.wait()
        lax.fori_loop(0, ngroups // 2, body, 0)
        copy_b.wait()
        lax.fori_loop(ngroups // 2, ngroups, body, 0)

        pltpu.sync_copy(w_v, w_hbm.at[pl.ds(tb0, tbpw)])
        pltpu.sync_copy(i_v, i_hbm.at[pl.ds(tb0, tbpw)])

    return router


@functools.cache
def _get_router():
    return _make_router()


def kernel(gate_logits):
    x4 = (gate_logits.T
          .reshape(NEB, EBLK, NTB, TBLK)
          .transpose(0, 2, 1, 3))
    w3, i3 = _get_router()(x4)
    weights = w3.transpose(0, 2, 1).reshape(NUM_TOKENS, TOPK)
    indices = i3.transpose(0, 2, 1).reshape(NUM_TOKENS, TOPK)
    return (weights, indices)


# speedup vs baseline: 3.8110x; 1.0021x over previous
import functools

import jax
import jax.numpy as jnp
from jax import lax
from jax.experimental import pallas as pl
from jax.experimental.pallas import tpu as pltpu
from jax.experimental.pallas import tpu_sc as plsc

NUM_TOKENS = 32768
NUM_EXPERTS = 64
TOPK = 8
LANES = 16
TBLK = 128
EBLK = 8
NEB = NUM_EXPERTS // EBLK
NTB = NUM_TOKENS // TBLK

_NET8 = (
    (0, 1), (2, 3), (4, 5), (6, 7),
    (0, 2), (1, 3), (4, 6), (5, 7),
    (1, 2), (5, 6), (0, 4), (3, 7),
    (1, 5), (2, 6),
    (1, 4), (3, 6),
    (2, 4), (3, 5),
    (3, 4),
)
_BMERGE8 = (
    (0, 4), (1, 5), (2, 6), (3, 7),
    (0, 2), (1, 3), (4, 6), (5, 7),
    (0, 1), (2, 3), (4, 5), (6, 7),
)


def _top8_keys(keys):
    acc = None
    for g in range(8):
        gv = list(keys[8 * g:8 * g + 8])
        for (i, j) in _NET8:
            hi = jnp.maximum(gv[i], gv[j])
            lo = jnp.minimum(gv[i], gv[j])
            gv[i], gv[j] = hi, lo
        if acc is None:
            acc = gv
        else:
            lv = [jnp.maximum(acc[i], gv[7 - i]) for i in range(8)]
            for (i, j) in _BMERGE8:
                hi = jnp.maximum(lv[i], lv[j])
                lo = jnp.minimum(lv[i], lv[j])
                lv[i], lv[j] = hi, lo
            acc = lv
    return acc


def _make_router():
    info = plsc.get_sparse_core_info()
    nc, ns = info.num_cores, info.num_subcores
    nw = nc * ns
    tpw = NUM_TOKENS // nw
    tbpw = tpw // TBLK
    ngroups = tpw // LANES

    mesh = plsc.VectorSubcoreMesh(core_axis_name="c", subcore_axis_name="s")

    @functools.partial(
        pl.kernel,
        mesh=mesh,
        out_type=(
            jax.ShapeDtypeStruct((NTB, TOPK, TBLK), jnp.float32),
            jax.ShapeDtypeStruct((NTB, TOPK, TBLK), jnp.int32),
        ),
        scratch_types=[
            pltpu.VMEM((NEB, tbpw, EBLK, TBLK), jnp.float32),
            pltpu.VMEM((tbpw, TOPK, TBLK), jnp.float32),
            pltpu.VMEM((tbpw, TOPK, TBLK), jnp.int32),
            pltpu.SemaphoreType.DMA,
            pltpu.SemaphoreType.DMA,
            pltpu.SemaphoreType.DMA,
        ],
        compiler_params=pltpu.CompilerParams(
            needs_layout_passes=False, use_tc_tiling_on_sc=False),
    )
    def router(logits_hbm, w_hbm, i_hbm, logits_v, w_v, i_v,
               sem_a, sem_b, sem_out):
        wid = lax.axis_index("s") * nc + lax.axis_index("c")
        tb0 = wid * tbpw
        half = tbpw // 2
        copy_a = pltpu.async_copy(
            logits_hbm.at[:, pl.ds(tb0, half)],
            logits_v.at[:, pl.ds(0, half)], sem_a)
        copy_b = pltpu.async_copy(
            logits_hbm.at[:, pl.ds(tb0 + half, half)],
            logits_v.at[:, pl.ds(half, half)], sem_b)

        def group(tb, gg):
            tr0 = gg * LANES
            keys = []
            for e in range(NUM_EXPERTS):
                v = logits_v[e // EBLK, tb, e % EBLK, pl.ds(tr0, LANES)]
                kb = (lax.bitcast_convert_type(v, jnp.int32)
                      & jnp.int32(-64)) | jnp.int32(63 - e)
                keys.append(lax.bitcast_convert_type(kb, jnp.float32))
            top = _top8_keys(keys)
            ids = [63 - (lax.bitcast_convert_type(k, jnp.int32)
                         & jnp.int32(63))
                   for k in top]
            es = ([jnp.full((LANES,), 1.0, jnp.float32)]
                  + [jnp.exp(v - top[0]) for v in top[1:]])
            total = es[0]
            for j in range(1, TOPK):
                total = total + es[j]
            inv = 1.0 / total
            for j in range(TOPK):
                w_v[tb, j, pl.ds(tr0, LANES)] = es[j] * inv
                i_v[tb, j, pl.ds(tr0, LANES)] = ids[j]

        def tb_body(tb, carry):
            def g_body(gg, c2):
                group(tb, gg)
                return c2
            lax.fori_loop(0, TBLK // LANES, g_body, 0)
            pltpu.async_copy(w_v.at[pl.ds(tb, 1)],
                             w_hbm.at[pl.ds(tb0 + tb, 1)], sem_out)
            pltpu.async_copy(i_v.at[pl.ds(tb, 1)],
                             i_hbm.at[pl.ds(tb0 + tb, 1)], sem_out)
            return carry

        copy_a.wait()
        lax.fori_loop(0, half, tb_body, 0)
        copy_b.wait()
        lax.fori_loop(half, tbpw, tb_body, 0)

        for t in range(tbpw):
            pltpu.make_async_copy(
                w_hbm.at[pl.ds(tb0 + t, 1)], w_v.at[pl.ds(t, 1)],
                sem_out).wait()
            pltpu.make_async_copy(
                i_hbm.at[pl.ds(tb0 + t, 1)], i_v.at[pl.ds(t, 1)],
                sem_out).wait()

    return router


@functools.cache
def _get_router():
    return _make_router()


def kernel(gate_logits):
    x4 = (gate_logits.T
          .reshape(NEB, EBLK, NTB, TBLK)
          .transpose(0, 2, 1, 3))
    w3, i3 = _get_router()(x4)
    weights = w3.transpose(0, 2, 1).reshape(NUM_TOKENS, TOPK)
    indices = i3.transpose(0, 2, 1).reshape(NUM_TOKENS, TOPK)
    return (weights, indices)


# speedup vs baseline: 3.8459x; 1.0092x over previous
import functools

import jax
import jax.numpy as jnp
from jax import lax
from jax.experimental import pallas as pl
from jax.experimental.pallas import tpu as pltpu
from jax.experimental.pallas import tpu_sc as plsc

NUM_TOKENS = 32768
NUM_EXPERTS = 64
TOPK = 8
LANES = 16
TBLK = 128
EBLK = 8
NEB = NUM_EXPERTS // EBLK
NTB = NUM_TOKENS // TBLK

_NET8 = (
    (0, 1), (2, 3), (4, 5), (6, 7),
    (0, 2), (1, 3), (4, 6), (5, 7),
    (1, 2), (5, 6), (0, 4), (3, 7),
    (1, 5), (2, 6),
    (1, 4), (3, 6),
    (2, 4), (3, 5),
    (3, 4),
)
_BMERGE8 = (
    (0, 4), (1, 5), (2, 6), (3, 7),
    (0, 2), (1, 3), (4, 6), (5, 7),
    (0, 1), (2, 3), (4, 5), (6, 7),
)


def _top8_keys(keys):
    acc = None
    for g in range(8):
        gv = list(keys[8 * g:8 * g + 8])
        for (i, j) in _NET8:
            hi = jnp.maximum(gv[i], gv[j])
            lo = jnp.minimum(gv[i], gv[j])
            gv[i], gv[j] = hi, lo
        if acc is None:
            acc = gv
        else:
            lv = [jnp.maximum(acc[i], gv[7 - i]) for i in range(8)]
            for (i, j) in _BMERGE8:
                hi = jnp.maximum(lv[i], lv[j])
                lo = jnp.minimum(lv[i], lv[j])
                lv[i], lv[j] = hi, lo
            acc = lv
    return acc


def _make_router():
    info = plsc.get_sparse_core_info()
    nc, ns = info.num_cores, info.num_subcores
    nw = nc * ns
    tpw = NUM_TOKENS // nw
    tbpw = tpw // TBLK
    ngroups = tpw // LANES

    mesh = plsc.VectorSubcoreMesh(core_axis_name="c", subcore_axis_name="s")

    @functools.partial(
        pl.kernel,
        mesh=mesh,
        out_type=(
            jax.ShapeDtypeStruct((NTB, TOPK, TBLK), jnp.float32),
            jax.ShapeDtypeStruct((NTB, TOPK, TBLK), jnp.int32),
        ),
        scratch_types=[
            pltpu.VMEM((NEB, tbpw, EBLK, TBLK), jnp.float32),
            pltpu.VMEM((tbpw, TOPK, TBLK), jnp.float32),
            pltpu.VMEM((tbpw, TOPK, TBLK), jnp.int32),
            pltpu.SemaphoreType.DMA,
            pltpu.SemaphoreType.DMA,
            pltpu.SemaphoreType.DMA,
        ],
        compiler_params=pltpu.CompilerParams(
            needs_layout_passes=False, use_tc_tiling_on_sc=False),
    )
    def router(logits_hbm, w_hbm, i_hbm, logits_v, w_v, i_v,
               sem_a, sem_b, sem_out):
        wid = lax.axis_index("s") * nc + lax.axis_index("c")
        tb0 = wid * tbpw
        half = tbpw // 2
        copy_a = pltpu.async_copy(
            logits_hbm.at[:, pl.ds(tb0, half)],
            logits_v.at[:, pl.ds(0, half)], sem_a)
        copy_b = pltpu.async_copy(
            logits_hbm.at[:, pl.ds(tb0 + half, half)],
            logits_v.at[:, pl.ds(half, half)], sem_b)

        def group(tb, gg):
            tr0 = gg * LANES
            keys = []
            for e in range(NUM_EXPERTS):
                v = logits_v[e // EBLK, tb, e % EBLK, pl.ds(tr0, LANES)]
                kb = (lax.bitcast_convert_type(v, jnp.int32)
                      & jnp.int32(-64)) | jnp.int32(63 - e)
                keys.append(lax.bitcast_convert_type(kb, jnp.float32))
            top = _top8_keys(keys)
            ids = [63 - (lax.bitcast_convert_type(k, jnp.int32)
                         & jnp.int32(63))
                   for k in top]
            es = ([jnp.full((LANES,), 1.0, jnp.float32)]
                  + [jnp.exp(v - top[0]) for v in top[1:]])
            total = es[0]
            for j in range(1, TOPK):
                total = total + es[j]
            inv = 1.0 / total
            for j in range(TOPK):
                w_v[tb, j, pl.ds(tr0, LANES)] = es[j] * inv
                i_v[tb, j, pl.ds(tr0, LANES)] = ids[j]

        def tb_body(tb, carry):
            @plsc.parallel_loop(0, TBLK // LANES, unroll=2)
            def g_body(gg):
                group(tb, gg)
            pltpu.async_copy(w_v.at[pl.ds(tb, 1)],
                             w_hbm.at[pl.ds(tb0 + tb, 1)], sem_out)
            pltpu.async_copy(i_v.at[pl.ds(tb, 1)],
                             i_hbm.at[pl.ds(tb0 + tb, 1)], sem_out)
            return carry

        copy_a.wait()
        lax.fori_loop(0, half, tb_body, 0)
        copy_b.wait()
        lax.fori_loop(half, tbpw, tb_body, 0)

        for t in range(tbpw):
            pltpu.make_async_copy(
                w_hbm.at[pl.ds(tb0 + t, 1)], w_v.at[pl.ds(t, 1)],
                sem_out).wait()
            pltpu.make_async_copy(
                i_hbm.at[pl.ds(tb0 + t, 1)], i_v.at[pl.ds(t, 1)],
                sem_out).wait()

    return router


@functools.cache
def _get_router():
    return _make_router()


def kernel(gate_logits):
    x4 = (gate_logits.T
          .reshape(NEB, EBLK, NTB, TBLK)
          .transpose(0, 2, 1, 3))
    w3, i3 = _get_router()(x4)
    weights = w3.transpose(0, 2, 1).reshape(NUM_TOKENS, TOPK)
    indices = i3.transpose(0, 2, 1).reshape(NUM_TOKENS, TOPK)
    return (weights, indices)
